# baseline (speedup 1.0000x reference)
import functools

import jax
import jax.numpy as jnp
from jax import lax
from jax.experimental import pallas as pl
from jax.experimental.pallas import tpu as pltpu
from jax.experimental.pallas import tpu_sc as plsc

D = 64
NC, NS = 2, 16
NW = NC * NS


def _sc_gather_score(u_emb, v_emb, uvidx, dur_w, seg, bpw, dur_slot):
    mesh = plsc.VectorSubcoreMesh(
        core_axis_name="c", subcore_axis_name="s", num_cores=NC, num_subcores=NS
    )

    @functools.partial(
        pl.kernel,
        out_type=jax.ShapeDtypeStruct((NW, 8, bpw), jnp.float32),
        mesh=mesh,
        compiler_params=pltpu.CompilerParams(
            use_tc_tiling_on_sc=True, needs_layout_passes=False),
        scratch_types=[
            pltpu.VMEM((seg, bpw), jnp.int32),
            pltpu.VMEM((seg, bpw, D), jnp.float32),
            pltpu.VMEM((1, D), jnp.float32),
            pltpu.VMEM((8, bpw), jnp.float32),
            pltpu.SemaphoreType.DMA,
        ],
    )
    def sc_kernel(u_hbm, v_hbm, uvidx_hbm, w_hbm, out_hbm,
                  idxbuf, rows, wbuf, scores, sem):
        wid = lax.axis_index("s") * NC + lax.axis_index("c")
        pltpu.sync_copy(uvidx_hbm.at[wid], idxbuf)
        pltpu.sync_copy(w_hbm, wbuf)

        for s in range(seg):
            src = u_hbm if s == 0 else v_hbm

            def group(g, carry, s=s, src=src):
                vec = idxbuf[s, pl.ds(g * 16, 16)]
                for k in range(16):
                    pltpu.async_copy(
                        src.at[pl.ds(vec[k], 1), :],
                        rows.at[s].at[pl.ds(g * 16 + k, 1), :],
                        sem)
                return carry

            lax.fori_loop(0, bpw // 16, group, 0)

        for s in range(seg):
            src = u_hbm if s == 0 else v_hbm
            pltpu.make_async_copy(src.at[pl.ds(0, bpw), :], rows.at[s], sem).wait()

        iota16 = lax.iota(jnp.int32, 16)
        zeros16 = jnp.zeros((16,), jnp.float32)
        nds = seg - 1

        def score_group(g, carry):
            pos = iota16 + g * 16
            accs = [zeros16] * nds
            accd = zeros16
            for d16 in range(D // 16):
                wv = wbuf[0, pl.ds(d16 * 16, 16)]
                for dd in range(16):
                    dfull = jnp.full((16,), d16 * 16 + dd, jnp.int32)
                    ucol = plsc.load_gather(
                        rows, [jnp.zeros((16,), jnp.int32), pos, dfull])
                    cols = [ucol]
                    for s in range(1, seg):
                        cols.append(plsc.load_gather(
                            rows, [jnp.full((16,), s, jnp.int32), pos, dfull]))
                        accs[s - 1] = accs[s - 1] + ucol * cols[s]
                    accd = accd + cols[dur_slot] * wv[dd]
            for s in range(nds):
                scores[s, pl.ds(g * 16, 16)] = accs[s]
            scores[nds, pl.ds(g * 16, 16)] = accd
            return carry

        lax.fori_loop(0, bpw // 16, score_group, 0)
        pltpu.sync_copy(scores, out_hbm.at[wid])

    return sc_kernel(u_emb, v_emb, uvidx, dur_w)


def _tc_finish(scores, dur_b, bpw, seg):
    B = NW * bpw
    nneg = seg - 2

    def body(s_ref, b_ref, loss_ref, dur_ref):
        w = pl.program_id(0)
        s = jnp.clip(s_ref[0:1, :], -10.0, 10.0)
        tot = jnp.log1p(jnp.exp(-s))
        negs = jnp.clip(s_ref[1:1 + nneg, :], -10.0, 10.0)
        tot = tot + jnp.sum(jnp.log1p(jnp.exp(negs)), axis=0, keepdims=True)
        part = jnp.sum(tot)

        @pl.when(w == 0)
        def _init():
            loss_ref[0] = 0.0

        loss_ref[0] += part

        @pl.when(w == NW - 1)
        def _finish():
            loss_ref[0] = loss_ref[0] / B

        dur_ref[...] = jnp.expand_dims(s_ref[seg - 1:seg, :] + b_ref[0], 0)

    return pl.pallas_call(
        body,
        grid=(NW,),
        in_specs=[
            pl.BlockSpec((8, bpw), lambda w: (w, 0)),
            pl.BlockSpec(memory_space=pltpu.SMEM),
        ],
        out_specs=[
            pl.BlockSpec(memory_space=pltpu.SMEM),
            pl.BlockSpec((1, 1, bpw), lambda w: (w, 0, 0)),
        ],
        out_shape=[
            jax.ShapeDtypeStruct((1,), jnp.float32),
            jax.ShapeDtypeStruct((NW, 1, bpw), jnp.float32),
        ],
    )(scores, dur_b)


def kernel(pos_u, pos_v, neg_v, predict_fix, u_emb, v_emb, dur_w, dur_b):
    B = pos_u.shape[0]
    nneg = neg_v.shape[1]
    seg = 2 + nneg
    bpw = B // NW

    negt = jnp.transpose(neg_v.reshape(NW, bpw, nneg), (0, 2, 1))
    uvidx = jnp.concatenate(
        [pos_u.reshape(NW, 1, bpw), pos_v.reshape(NW, 1, bpw), negt], axis=1)

    dur_from_v = isinstance(predict_fix, str) and predict_fix == "output"
    scores = _sc_gather_score(u_emb, v_emb, uvidx, dur_w, seg, bpw,
                              1 if dur_from_v else 0)
    loss, dur = _tc_finish(scores.reshape(NW * 8, bpw), dur_b, bpw, seg)
    return loss[0], dur.reshape(B)


# speedup vs baseline: 1.0151x; 1.0151x over previous
import functools

import jax
import jax.numpy as jnp
from jax import lax
from jax.experimental import pallas as pl
from jax.experimental.pallas import tpu as pltpu
from jax.experimental.pallas import tpu_sc as plsc

D = 64
NC, NS = 2, 16
NW = NC * NS


def _sc_gather(u_emb, v_emb, uvidx, seg, bpw):
    mesh = plsc.VectorSubcoreMesh(
        core_axis_name="c", subcore_axis_name="s", num_cores=NC, num_subcores=NS
    )

    @functools.partial(
        pl.kernel,
        out_type=jax.ShapeDtypeStruct((NW, seg, bpw, D), jnp.float32),
        mesh=mesh,
        compiler_params=pltpu.CompilerParams(use_tc_tiling_on_sc=True),
        scratch_types=[
            pltpu.VMEM((seg, bpw), jnp.int32),
            pltpu.VMEM((seg, bpw, D), jnp.float32),
            pltpu.SemaphoreType.DMA,
        ],
    )
    def sc_kernel(u_hbm, v_hbm, uvidx_hbm, out_hbm, idxbuf, rows, sem):
        wid = lax.axis_index("s") * NC + lax.axis_index("c")
        pltpu.sync_copy(uvidx_hbm.at[wid], idxbuf)

        for s in range(seg):
            src = u_hbm if s == 0 else v_hbm

            def group(g, carry, s=s, src=src):
                vec = idxbuf[s, pl.ds(g * 16, 16)]
                for k in range(16):
                    pltpu.async_copy(
                        src.at[pl.ds(vec[k], 1), :],
                        rows.at[s].at[pl.ds(g * 16 + k, 1), :],
                        sem)
                return carry

            lax.fori_loop(0, bpw // 16, group, 0)

        for s in range(seg):
            src = u_hbm if s == 0 else v_hbm
            pltpu.make_async_copy(src.at[pl.ds(0, bpw), :], rows.at[s], sem).wait()

        pltpu.sync_copy(rows, out_hbm.at[wid])

    return sc_kernel(u_emb, v_emb, uvidx)


def _tc_score(rows, dur_w, dur_b, bpw, seg, dur_from_v):
    B = NW * bpw

    def body(v_ref, w_ref, b_ref, loss_ref, dur_ref):
        w = pl.program_id(0)
        u = v_ref[0:bpw, :]
        pv = v_ref[bpw:2 * bpw, :]
        s = jnp.clip(jnp.sum(u * pv, axis=1, keepdims=True), -10.0, 10.0)
        tot = jnp.log1p(jnp.exp(-s))
        for j in range(2, seg):
            nvr = v_ref[j * bpw:(j + 1) * bpw, :]
            ns = jnp.clip(jnp.sum(u * nvr, axis=1, keepdims=True), -10.0, 10.0)
            tot = tot + jnp.log1p(jnp.exp(ns))
        part = jnp.sum(tot)

        @pl.when(w == 0)
        def _init():
            loss_ref[0] = 0.0

        loss_ref[0] += part

        @pl.when(w == NW - 1)
        def _finish():
            loss_ref[0] = loss_ref[0] / B

        sel = pv if dur_from_v else u
        dur_ref[...] = jnp.sum(sel * w_ref[...], axis=1, keepdims=True) + b_ref[0]

    return pl.pallas_call(
        body,
        grid=(NW,),
        in_specs=[
            pl.BlockSpec((seg * bpw, D), lambda w: (w, 0)),
            pl.BlockSpec((1, D), lambda w: (0, 0)),
            pl.BlockSpec(memory_space=pltpu.SMEM),
        ],
        out_specs=[
            pl.BlockSpec(memory_space=pltpu.SMEM),
            pl.BlockSpec((bpw, 1), lambda w: (w, 0)),
        ],
        out_shape=[
            jax.ShapeDtypeStruct((1,), jnp.float32),
            jax.ShapeDtypeStruct((B, 1), jnp.float32),
        ],
    )(rows, dur_w, dur_b)


def kernel(pos_u, pos_v, neg_v, predict_fix, u_emb, v_emb, dur_w, dur_b):
    B = pos_u.shape[0]
    nneg = neg_v.shape[1]
    seg = 2 + nneg
    bpw = B // NW

    negt = jnp.transpose(neg_v.reshape(NW, bpw, nneg), (0, 2, 1))
    uvidx = jnp.concatenate(
        [pos_u.reshape(NW, 1, bpw), pos_v.reshape(NW, 1, bpw), negt], axis=1)

    rows = _sc_gather(u_emb, v_emb, uvidx, seg, bpw)

    dur_from_v = isinstance(predict_fix, str) and predict_fix == "output"
    loss, dur = _tc_score(
        rows.reshape(NW * seg * bpw, D), dur_w, dur_b, bpw, seg, dur_from_v)
    return loss[0], dur.reshape(B)
